# baseline (speedup 1.0000x reference)
import functools

import jax
import jax.numpy as jnp
from jax import lax
from jax.experimental import pallas as pl
from jax.experimental.pallas import tpu as pltpu
from jax.experimental.pallas import tpu_sc as plsc

D = 128
NC, NS = 2, 16
NW = NC * NS
NB = 2
K = 4
BLK = 16


def _pad8(n):
    return (n + 7) // 8 * 8


@functools.lru_cache(maxsize=None)
def _emb_call(batch: int, seq: int, vocab: int):
    assert batch % (NW * NB) == 0
    seq_pad0 = _pad8(seq)
    rows_per_chunk = NB * seq_pad0
    assert rows_per_chunk <= 128
    batches_per_w = batch // NW
    n_chunks = batches_per_w // NB
    seq_pad = _pad8(seq)

    NBUF = 4
    LOOK = 3
    assert n_chunks % NBUF == 0

    mesh = plsc.VectorSubcoreMesh(
        core_axis_name="c", subcore_axis_name="s",
        num_cores=NC, num_subcores=NS,
    )

    @functools.partial(
        pl.kernel,
        out_type=jax.ShapeDtypeStruct((batch, seq_pad, D), jnp.float32),
        mesh=mesh,
        scratch_types=[
            pltpu.VMEM((n_chunks, 128), jnp.int32),
            [pltpu.VMEM((rows_per_chunk, D), jnp.float32) for _ in range(NBUF)],
            [pltpu.SemaphoreType.DMA for _ in range(NBUF)],
            [pltpu.SemaphoreType.DMA for _ in range(NBUF)],
        ],
    )
    def emb(idx_hbm, table_hbm, out_hbm, idx_v, bufs, gsem, ssem):
        wid = lax.axis_index("s") * NC + lax.axis_index("c")
        base = wid * batches_per_w

        pltpu.sync_copy(idx_hbm.at[wid], idx_v)

        def fire_gather(c, b):
            pltpu.async_copy(
                table_hbm.at[idx_v.at[c, pl.ds(0, rows_per_chunk)]],
                bufs[b], gsem[b],
            )

        def fire_stores(c, b):
            for k in range(NB):
                pltpu.async_copy(
                    bufs[b].at[pl.ds(k * seq_pad, seq_pad)],
                    out_hbm.at[base + c * NB + k],
                    ssem[b],
                )

        def drain_stores(c, b):
            for k in range(NB):
                pltpu.make_async_copy(
                    bufs[b].at[pl.ds(k * seq_pad, seq_pad)],
                    out_hbm.at[base + c * NB + k],
                    ssem[b],
                ).wait()

        for b in range(LOOK):
            fire_gather(b, b)

        @pl.loop(0, n_chunks, step=NBUF)
        def _(j):
            for b in range(NBUF):
                c = j + b
                pltpu.make_async_copy(
                    table_hbm.at[idx_v.at[c, pl.ds(0, rows_per_chunk)]],
                    bufs[b], gsem[b],
                ).wait()
                fire_stores(c, b)

                bb = (b + LOOK) % NBUF

                @pl.when(c + LOOK < n_chunks)
                def _():
                    @pl.when(c >= NBUF - LOOK)
                    def _():
                        drain_stores(c, bb)

                    fire_gather(c + LOOK, bb)

        for b in range(NBUF):
            drain_stores(0, b)

    return emb


@functools.lru_cache(maxsize=None)
def _retile_call(batch: int, per: int, seq: int, off_blocks: int, first: bool):
    seq_pad = _pad8(seq)
    grid = (per // BLK,)

    def body(*refs):
        src_ref, out_ref = refs[-2], refs[-1]
        x = src_ref[...]
        out_ref[...] = jax.lax.slice(x, (0, 0, 0), (BLK, seq, D))

    in_specs = []
    if not first:
        in_specs.append(pl.BlockSpec(memory_space=pl.ANY))
    in_specs.append(pl.BlockSpec((BLK, seq_pad, D), lambda i: (i, 0, 0)))

    return pl.pallas_call(
        body,
        grid=grid,
        in_specs=in_specs,
        out_specs=pl.BlockSpec(
            (BLK, seq, D), lambda i: (i + off_blocks, 0, 0)
        ),
        out_shape=jax.ShapeDtypeStruct((batch, seq, D), jnp.float32),
        input_output_aliases={} if first else {0: 0},
    )


def kernel(token_ids, emb_table):
    batch, seq = token_ids.shape
    vocab, d = emb_table.shape
    assert d == D
    per = batch // K
    sc_call = _emb_call(per, seq, vocab)
    seq_pad = _pad8(seq)
    ids = token_ids.astype(jnp.int32).reshape(K, NW, per // NW // NB, NB, seq)
    ids = jnp.pad(ids, ((0, 0), (0, 0), (0, 0), (0, 0), (0, seq_pad - seq)))
    ids = ids.reshape(K, NW, -1, NB * seq_pad)
    ids = jnp.pad(ids, ((0, 0), (0, 0), (0, 0), (0, 128 - NB * seq_pad)))
    parts = [sc_call(ids[i], emb_table) for i in range(K)]

    out = _retile_call(batch, per, seq, 0, True)(parts[0])
    for i in range(1, K):
        out = _retile_call(batch, per, seq, i * per // BLK, False)(
            out, parts[i]
        )
    return out


# speedup vs baseline: 8.5519x; 8.5519x over previous
import functools

import jax
import jax.numpy as jnp
from jax import lax
from jax.experimental import pallas as pl
from jax.experimental.pallas import tpu as pltpu
from jax.experimental.pallas import tpu_sc as plsc

D = 128
NC, NS = 2, 16
NW = NC * NS
NB = 2


@functools.lru_cache(maxsize=None)
def _emb_call(batch: int, seq: int, vocab: int):
    assert batch % (NW * NB) == 0
    rows_per_chunk = NB * seq
    assert rows_per_chunk <= 128
    batches_per_w = batch // NW
    n_chunks = batches_per_w // NB

    NBUF = 4
    LOOK = 3
    assert n_chunks % NBUF == 0

    mesh = plsc.VectorSubcoreMesh(
        core_axis_name="c", subcore_axis_name="s",
        num_cores=NC, num_subcores=NS,
    )

    @functools.partial(
        pl.kernel,
        out_type=jax.ShapeDtypeStruct((batch, seq, D), jnp.float32),
        mesh=mesh,
        compiler_params=pltpu.CompilerParams(use_tc_tiling_on_sc=True),
        scratch_types=[
            pltpu.VMEM((n_chunks, 128), jnp.int32),
            [pltpu.VMEM((rows_per_chunk, D), jnp.float32) for _ in range(NBUF)],
            [pltpu.SemaphoreType.DMA for _ in range(NBUF)],
            [pltpu.SemaphoreType.DMA for _ in range(NBUF)],
        ],
    )
    def emb(idx_hbm, table_hbm, out_hbm, idx_v, bufs, gsem, ssem):
        wid = lax.axis_index("s") * NC + lax.axis_index("c")
        base = wid * batches_per_w

        pltpu.sync_copy(idx_hbm.at[wid], idx_v)

        def fire_gather(c, b):
            pltpu.async_copy(
                table_hbm.at[idx_v.at[c, pl.ds(0, rows_per_chunk)]],
                bufs[b], gsem[b],
            )

        def fire_stores(c, b):
            for k in range(NB):
                pltpu.async_copy(
                    bufs[b].at[pl.ds(k * seq, seq)],
                    out_hbm.at[base + c * NB + k],
                    ssem[b],
                )

        def drain_stores(c, b):
            for k in range(NB):
                pltpu.make_async_copy(
                    bufs[b].at[pl.ds(k * seq, seq)],
                    out_hbm.at[base + c * NB + k],
                    ssem[b],
                ).wait()

        for b in range(LOOK):
            fire_gather(b, b)

        @pl.loop(0, n_chunks, step=NBUF)
        def _(j):
            for b in range(NBUF):
                c = j + b
                pltpu.make_async_copy(
                    table_hbm.at[idx_v.at[c, pl.ds(0, rows_per_chunk)]],
                    bufs[b], gsem[b],
                ).wait()
                fire_stores(c, b)

                bb = (b + LOOK) % NBUF

                @pl.when(c + LOOK < n_chunks)
                def _():
                    @pl.when(c >= NBUF - LOOK)
                    def _():
                        drain_stores(c, bb)

                    fire_gather(c + LOOK, bb)

        for b in range(NBUF):
            drain_stores(0, b)

    return emb


def kernel(token_ids, emb_table):
    batch, seq = token_ids.shape
    vocab, d = emb_table.shape
    assert d == D
    ids = token_ids.astype(jnp.int32).reshape(NW, -1, NB * seq)
    ids = jnp.pad(ids, ((0, 0), (0, 0), (0, 128 - NB * seq)))
    return _emb_call(batch, seq, vocab)(ids, emb_table)
